# baseline (speedup 1.0000x reference)
import functools

import jax
import jax.numpy as jnp
from jax import lax
from jax.experimental import pallas as pl
from jax.experimental.pallas import tpu as pltpu
import jax.experimental.pallas.tpu_sc as plsc

B = 4
N = 8192
K = 2048
D = 768
ROWS = N // 128
KROWS = K // 128

_NC = 2
_NS = 16
_NW = _NC * _NS
_CHUNK = K // _NW


def _sort_body(sig_ref, out_ref):
    b = pl.program_id(0)
    v = sig_ref[0]
    row = lax.broadcasted_iota(jnp.int32, (ROWS, 128), 0)
    col = lax.broadcasted_iota(jnp.int32, (ROWS, 128), 1)
    flat = row * 128 + col
    idx = flat

    k = 2
    while k <= N:
        j = k // 2
        while j >= 1:
            is_lo = (flat & j) == 0
            desc = (flat & k) == 0
            if j >= 128:
                s = j // 128
                up_v = jnp.roll(v, -s, axis=0)
                dn_v = jnp.roll(v, s, axis=0)
                up_i = jnp.roll(idx, -s, axis=0)
                dn_i = jnp.roll(idx, s, axis=0)
            else:
                up_v = jnp.roll(v, -j, axis=1)
                dn_v = jnp.roll(v, j, axis=1)
                up_i = jnp.roll(idx, -j, axis=1)
                dn_i = jnp.roll(idx, j, axis=1)
            pv = jnp.where(is_lo, up_v, dn_v)
            pi = jnp.where(is_lo, up_i, dn_i)
            sg = (v > pv) | ((v == pv) & (idx < pi))
            ts = sg ^ is_lo ^ desc
            v = jnp.where(ts, v, pv)
            idx = jnp.where(ts, idx, pi)
            j //= 2
        k *= 2

    del b
    out_ref[0] = idx[0:KROWS, :] + 1


def _topk_indices(significance):
    sig3 = significance.reshape(B, ROWS, 128)
    out = pl.pallas_call(
        _sort_body,
        grid=(B,),
        in_specs=[pl.BlockSpec((1, ROWS, 128), lambda b: (b, 0, 0))],
        out_specs=pl.BlockSpec((1, KROWS, 128), lambda b: (b, 0, 0)),
        out_shape=jax.ShapeDtypeStruct((B, KROWS, 128), jnp.int32),
    )(sig3)
    return out.reshape(B * K)


_SEG = 17 * 128
_CW = 128
_NCHUNK = B * (K // _CW)
_PER_W = _NCHUNK // _NW


def _gather_body(x_hbm, idx_hbm, out_hbm, idx_v, rows_v, idx1_v, row1_v, sem):
    wid = lax.axis_index("s") * _NC + lax.axis_index("c")

    for u in range(_PER_W):
        gc = wid * _PER_W + u
        b = gc // (K // _CW)
        ci = gc % (K // _CW)
        pltpu.sync_copy(idx_hbm.at[pl.ds(b * _SEG + ci * _CW, _CW)], idx_v)
        pltpu.async_copy(x_hbm.at[b].at[idx_v], rows_v, sem).wait()
        pltpu.sync_copy(rows_v, out_hbm.at[b, pl.ds(ci * _CW, _CW)])

    @pl.when(wid < B)
    def _tail():
        pltpu.sync_copy(idx_hbm.at[pl.ds(wid * _SEG + K, 1)], idx1_v)
        pltpu.async_copy(x_hbm.at[wid].at[idx1_v], row1_v, sem).wait()
        pltpu.sync_copy(row1_v, out_hbm.at[wid, pl.ds(K, 1)])


@functools.cache
def _gather_call():
    return functools.partial(
        pl.kernel,
        out_type=jax.ShapeDtypeStruct((B, K + 1, D), jnp.float32),
        mesh=plsc.VectorSubcoreMesh(
            core_axis_name="c", subcore_axis_name="s",
            num_cores=_NC, num_subcores=_NS),
        scratch_types=[
            pltpu.VMEM((_CW,), jnp.int32),
            pltpu.VMEM((_CW, D), jnp.float32),
            pltpu.VMEM((1,), jnp.int32),
            pltpu.VMEM((1, D), jnp.float32),
            pltpu.SemaphoreType.DMA,
        ],
        compiler_params=pltpu.CompilerParams(
            use_tc_tiling_on_sc=True, needs_layout_passes=True),
    )(_gather_body)


def kernel(x, significance):
    idx = _topk_indices(significance).reshape(B, K)
    cls_src = jnp.zeros((B, 1), dtype=jnp.int32)
    pad = jnp.zeros((B, _SEG - (K + 1)), dtype=jnp.int32)
    src_idx = jnp.concatenate([cls_src, idx, pad], axis=1).reshape(B * _SEG)
    return _gather_call()(x, src_idx)


# speedup vs baseline: 1.7972x; 1.7972x over previous
import functools

import jax
import jax.numpy as jnp
from jax import lax
from jax.experimental import pallas as pl
from jax.experimental.layout import Layout, with_layout_constraint
from jax.experimental.pallas import tpu as pltpu
import jax.experimental.pallas.tpu_sc as plsc

B = 4
N = 8192
K = 2048
D = 768
ROWS = N // 128
KROWS = K // 128

_NC = 2
_NS = 16
_NW = _NC * _NS
_CHUNK = K // _NW


def _sort_body(sig_ref, out_ref):
    b = pl.program_id(0)
    v = sig_ref[0]
    row = lax.broadcasted_iota(jnp.int32, (ROWS, 128), 0)
    col = lax.broadcasted_iota(jnp.int32, (ROWS, 128), 1)
    flat = row * 128 + col
    idx = flat

    k = 2
    while k <= N:
        j = k // 2
        while j >= 1:
            is_lo = (flat & j) == 0
            desc = (flat & k) == 0
            if j >= 128:
                s = j // 128
                up_v = jnp.roll(v, -s, axis=0)
                dn_v = jnp.roll(v, s, axis=0)
                up_i = jnp.roll(idx, -s, axis=0)
                dn_i = jnp.roll(idx, s, axis=0)
            else:
                up_v = jnp.roll(v, -j, axis=1)
                dn_v = jnp.roll(v, j, axis=1)
                up_i = jnp.roll(idx, -j, axis=1)
                dn_i = jnp.roll(idx, j, axis=1)
            pv = jnp.where(is_lo, up_v, dn_v)
            pi = jnp.where(is_lo, up_i, dn_i)
            sg = (v > pv) | ((v == pv) & (idx < pi))
            ts = sg ^ is_lo ^ desc
            v = jnp.where(ts, v, pv)
            idx = jnp.where(ts, idx, pi)
            j //= 2
        k *= 2

    del b
    out_ref[0] = idx[0:KROWS, :] + 1


def _topk_indices(significance):
    sig3 = significance.reshape(B, ROWS, 128)
    out = pl.pallas_call(
        _sort_body,
        grid=(B,),
        in_specs=[pl.BlockSpec((1, ROWS, 128), lambda b: (b, 0, 0))],
        out_specs=pl.BlockSpec((1, KROWS, 128), lambda b: (b, 0, 0)),
        out_shape=jax.ShapeDtypeStruct((B, KROWS, 128), jnp.int32),
    )(sig3)
    return out.reshape(B * K)


_JT = D // 128
_SUB = _JT * B
_TOT = (K + 1) * _SUB
_CWS = 128
_FULL = (_TOT // _CWS) * _CWS
_PER_W = (_FULL // _CWS) // _NW
_TAIL = _TOT - _FULL


def _gather_body(x_hbm, idx_hbm, out_hbm, idx_v, rows_v, idxt_v, rowst_v, sem):
    wid = lax.axis_index("s") * _NC + lax.axis_index("c")

    for u in range(_PER_W):
        off = (wid * _PER_W + u) * _CWS
        pltpu.sync_copy(idx_hbm.at[pl.ds(off, _CWS)], idx_v)
        pltpu.async_copy(x_hbm.at[idx_v], rows_v, sem).wait()
        pltpu.sync_copy(rows_v, out_hbm.at[pl.ds(off, _CWS)])

    @pl.when(wid == _NW - 1)
    def _tail():
        pltpu.sync_copy(idx_hbm.at[pl.ds(_FULL, _TAIL)], idxt_v)
        pltpu.async_copy(x_hbm.at[idxt_v], rowst_v, sem).wait()
        pltpu.sync_copy(rowst_v, out_hbm.at[pl.ds(_FULL, _TAIL)])


@functools.cache
def _gather_call():
    return functools.partial(
        pl.kernel,
        out_type=jax.ShapeDtypeStruct((_TOT, 128), jnp.float32),
        mesh=plsc.VectorSubcoreMesh(
            core_axis_name="c", subcore_axis_name="s",
            num_cores=_NC, num_subcores=_NS),
        scratch_types=[
            pltpu.VMEM((_CWS,), jnp.int32),
            pltpu.VMEM((_CWS, 128), jnp.float32),
            pltpu.VMEM((_TAIL,), jnp.int32),
            pltpu.VMEM((_TAIL, 128), jnp.float32),
            pltpu.SemaphoreType.DMA,
        ],
    )(_gather_body)


def kernel(x, significance):
    tok = _topk_indices(significance).reshape(B, K)
    tok = jnp.concatenate([jnp.zeros((B, 1), jnp.int32), tok], axis=1)
    sub = (tok.T[:, None, :] * _SUB
           + (jnp.arange(_JT, dtype=jnp.int32) * B)[None, :, None]
           + jnp.arange(B, dtype=jnp.int32)[None, None, :])
    src_idx = sub.reshape(_TOT)
    xa = with_layout_constraint(
        x.transpose(1, 0, 2),
        Layout(major_to_minor=(0, 1, 2), tiling=((4, 128),)))
    xb = with_layout_constraint(
        xa.reshape(N + 1, B, _JT, 128),
        Layout(major_to_minor=(0, 2, 1, 3), tiling=((4, 128),)))
    xc = with_layout_constraint(
        xb.transpose(0, 2, 1, 3),
        Layout(major_to_minor=(0, 1, 2, 3), tiling=((4, 128),)))
    x2 = xc.reshape((N + 1) * _SUB, 128)
    outz = _gather_call()(x2, src_idx)
    oe = with_layout_constraint(
        outz.reshape(K + 1, _JT, B, 128),
        Layout(major_to_minor=(0, 1, 2, 3), tiling=((4, 128),)))
    of = with_layout_constraint(
        oe.transpose(2, 0, 1, 3),
        Layout(major_to_minor=(1, 2, 0, 3), tiling=((4, 128),)))
    return of.reshape(B, K + 1, D)


# speedup vs baseline: 2.0857x; 1.1605x over previous
import functools

import jax
import jax.numpy as jnp
from jax import lax
from jax.experimental import pallas as pl
from jax.experimental.layout import Layout, with_layout_constraint
from jax.experimental.pallas import tpu as pltpu
import jax.experimental.pallas.tpu_sc as plsc

B = 4
N = 8192
K = 2048
D = 768
ROWS = N // 128
KROWS = K // 128

_NC = 2
_NS = 16
_NW = _NC * _NS
_CHUNK = K // _NW


GROWS = B * ROWS


def _sort_body(sig_ref, out_ref):
    v = sig_ref[...]
    row = lax.broadcasted_iota(jnp.int32, (GROWS, 128), 0)
    col = lax.broadcasted_iota(jnp.int32, (GROWS, 128), 1)
    flat = (row % ROWS) * 128 + col
    idx = flat

    k = 2
    while k <= N:
        j = k // 2
        while j >= 1:
            is_lo = (flat & j) == 0
            desc = (flat & k) == 0
            if j >= 128:
                s = j // 128
                up_v = jnp.roll(v, -s, axis=0)
                dn_v = jnp.roll(v, s, axis=0)
                up_i = jnp.roll(idx, -s, axis=0)
                dn_i = jnp.roll(idx, s, axis=0)
            else:
                up_v = jnp.roll(v, -j, axis=1)
                dn_v = jnp.roll(v, j, axis=1)
                up_i = jnp.roll(idx, -j, axis=1)
                dn_i = jnp.roll(idx, j, axis=1)
            pv = jnp.where(is_lo, up_v, dn_v)
            pi = jnp.where(is_lo, up_i, dn_i)
            sg = (v > pv) | ((v == pv) & (idx < pi))
            ts = sg ^ is_lo ^ desc
            v = jnp.where(ts, v, pv)
            idx = jnp.where(ts, idx, pi)
            j //= 2
        k *= 2

    out_ref[...] = jnp.concatenate(
        [idx[b * ROWS:b * ROWS + KROWS, :] for b in range(B)], axis=0) + 1


def _topk_indices(significance):
    sig2 = significance.reshape(GROWS, 128)
    out = pl.pallas_call(
        _sort_body,
        out_shape=jax.ShapeDtypeStruct((B * KROWS, 128), jnp.int32),
    )(sig2)
    return out.reshape(B * K)


_JT = D // 128
_SUB = _JT * B
_TOT = (K + 1) * _SUB
_CWS = 128
_FULL = (_TOT // _CWS) * _CWS
_PER_W = (_FULL // _CWS) // _NW
_TAIL = _TOT - _FULL


def _gather_body(x_hbm, idx_hbm, out_hbm, idx_v, rows_v, idxt_v, rowst_v, sem):
    wid = lax.axis_index("s") * _NC + lax.axis_index("c")

    for u in range(_PER_W):
        off = (wid * _PER_W + u) * _CWS
        pltpu.sync_copy(idx_hbm.at[pl.ds(off, _CWS)], idx_v)
        pltpu.async_copy(x_hbm.at[idx_v], rows_v, sem).wait()
        pltpu.sync_copy(rows_v, out_hbm.at[pl.ds(off, _CWS)])

    @pl.when(wid == _NW - 1)
    def _tail():
        pltpu.sync_copy(idx_hbm.at[pl.ds(_FULL, _TAIL)], idxt_v)
        pltpu.async_copy(x_hbm.at[idxt_v], rowst_v, sem).wait()
        pltpu.sync_copy(rowst_v, out_hbm.at[pl.ds(_FULL, _TAIL)])


@functools.cache
def _gather_call():
    return functools.partial(
        pl.kernel,
        out_type=jax.ShapeDtypeStruct((_TOT, 128), jnp.float32),
        mesh=plsc.VectorSubcoreMesh(
            core_axis_name="c", subcore_axis_name="s",
            num_cores=_NC, num_subcores=_NS),
        scratch_types=[
            pltpu.VMEM((_CWS,), jnp.int32),
            pltpu.VMEM((_CWS, 128), jnp.float32),
            pltpu.VMEM((_TAIL,), jnp.int32),
            pltpu.VMEM((_TAIL, 128), jnp.float32),
            pltpu.SemaphoreType.DMA,
        ],
    )(_gather_body)


def kernel(x, significance):
    tok = _topk_indices(significance).reshape(B, K)
    tok = jnp.concatenate([jnp.zeros((B, 1), jnp.int32), tok], axis=1)
    sub = (tok.T[:, None, :] * _SUB
           + (jnp.arange(_JT, dtype=jnp.int32) * B)[None, :, None]
           + jnp.arange(B, dtype=jnp.int32)[None, None, :])
    src_idx = sub.reshape(_TOT)
    xa = with_layout_constraint(
        x.transpose(1, 0, 2),
        Layout(major_to_minor=(0, 1, 2), tiling=((4, 128),)))
    xb = with_layout_constraint(
        xa.reshape(N + 1, B, _JT, 128),
        Layout(major_to_minor=(0, 2, 1, 3), tiling=((4, 128),)))
    xc = with_layout_constraint(
        xb.transpose(0, 2, 1, 3),
        Layout(major_to_minor=(0, 1, 2, 3), tiling=((4, 128),)))
    x2 = xc.reshape((N + 1) * _SUB, 128)
    outz = _gather_call()(x2, src_idx)
    oe = with_layout_constraint(
        outz.reshape(K + 1, _JT, B, 128),
        Layout(major_to_minor=(0, 1, 2, 3), tiling=((4, 128),)))
    of = with_layout_constraint(
        oe.transpose(2, 0, 1, 3),
        Layout(major_to_minor=(1, 2, 0, 3), tiling=((4, 128),)))
    return of.reshape(B, K + 1, D)


# speedup vs baseline: 2.7813x; 1.3335x over previous
import functools

import jax
import jax.numpy as jnp
from jax import lax
from jax.experimental import pallas as pl
from jax.experimental.layout import Layout, with_layout_constraint
from jax.experimental.pallas import tpu as pltpu
import jax.experimental.pallas.tpu_sc as plsc

B = 4
N = 8192
K = 2048
D = 768
ROWS = N // 128
KROWS = K // 128

_NC = 2
_NS = 16
_NW = _NC * _NS
_CHUNK = K // _NW


GROWS = B * ROWS


def _sort_body(sig_ref, out_ref):
    v = sig_ref[...]
    row = lax.broadcasted_iota(jnp.int32, (GROWS, 128), 0)
    col = lax.broadcasted_iota(jnp.int32, (GROWS, 128), 1)
    flat = (row % ROWS) * 128 + col
    idx = flat

    k = 2
    while k <= N:
        j = k // 2
        while j >= 1:
            is_lo = (flat & j) == 0
            desc = (flat & k) == 0
            if j >= 128:
                s = j // 128
                up_v = jnp.roll(v, -s, axis=0)
                dn_v = jnp.roll(v, s, axis=0)
                up_i = jnp.roll(idx, -s, axis=0)
                dn_i = jnp.roll(idx, s, axis=0)
            else:
                up_v = jnp.roll(v, -j, axis=1)
                dn_v = jnp.roll(v, j, axis=1)
                up_i = jnp.roll(idx, -j, axis=1)
                dn_i = jnp.roll(idx, j, axis=1)
            pv = jnp.where(is_lo, up_v, dn_v)
            pi = jnp.where(is_lo, up_i, dn_i)
            sg = (v > pv) | ((v == pv) & (idx < pi))
            ts = sg ^ is_lo ^ desc
            v = jnp.where(ts, v, pv)
            idx = jnp.where(ts, idx, pi)
            j //= 2
        k *= 2

    out_ref[...] = jnp.concatenate(
        [idx[b * ROWS:b * ROWS + KROWS, :] for b in range(B)], axis=0) + 1


def _topk_indices(significance):
    sig2 = significance.reshape(GROWS, 128)
    out = pl.pallas_call(
        _sort_body,
        out_shape=jax.ShapeDtypeStruct((B * KROWS, 128), jnp.int32),
    )(sig2)
    return out.reshape(B * K)


_JT = D // 128
_SUB = _JT * B
_TOT = (K + 1) * _SUB
_CWS = 128
_FULL = (_TOT // _CWS) * _CWS
_PER_W = (_FULL // _CWS) // _NW
_TAIL = _TOT - _FULL
_PP = 2056
_L = 16


def _div24(svec):
    return lax.shift_right_logical(svec * 43691, 20)


def _expand_idx(tok_v, idx_v, buf, off):
    for g in range(_CWS // _L):
        svec = lax.iota(jnp.int32, _L) + (off + g * _L)
        pvec = _div24(svec)
        rvec = svec - pvec * _SUB
        t = plsc.load_gather(tok_v, [pvec * B + (rvec & (B - 1))])
        idx_v[buf, pl.ds(g * _L, _L)] = t * _SUB + rvec


def _gather_body(x_hbm, tok_hbm, out_hbm, tok_v, idx_v, rows_v,
                 idxt_v, rowst_v, sem0, sem1):
    wid = lax.axis_index("s") * _NC + lax.axis_index("c")
    sems = (sem0, sem1)
    pltpu.sync_copy(tok_hbm, tok_v)

    def chunk_off(u):
        return (wid * _PER_W + u) * _CWS

    _expand_idx(tok_v, idx_v, 0, chunk_off(0))
    descs = {0: pltpu.async_copy(x_hbm.at[idx_v.at[0]], rows_v.at[0], sem0)}
    for u in range(_PER_W):
        cb = u % 2
        if u + 1 < _PER_W:
            nb = (u + 1) % 2
            _expand_idx(tok_v, idx_v, nb, chunk_off(u + 1))
            descs[u + 1] = pltpu.async_copy(
                x_hbm.at[idx_v.at[nb]], rows_v.at[nb], sems[nb])
        descs[u].wait()
        pltpu.sync_copy(rows_v.at[cb], out_hbm.at[pl.ds(chunk_off(u), _CWS)])

    @pl.when(wid == _NW - 1)
    def _tail():
        for g in range(2):
            svec = lax.iota(jnp.int32, _L) + (_FULL + g * _L)
            pvec = _div24(svec)
            rvec = svec - pvec * _SUB
            t = plsc.load_gather(tok_v, [pvec * B + (rvec & (B - 1))])
            idxt_v[pl.ds(g * _L, _L)] = t * _SUB + rvec
        pltpu.async_copy(x_hbm.at[idxt_v], rowst_v, sem0).wait()
        pltpu.sync_copy(rowst_v.at[pl.ds(0, _TAIL)],
                        out_hbm.at[pl.ds(_FULL, _TAIL)])


@functools.cache
def _gather_call():
    return functools.partial(
        pl.kernel,
        out_type=jax.ShapeDtypeStruct((_TOT, 128), jnp.float32),
        mesh=plsc.VectorSubcoreMesh(
            core_axis_name="c", subcore_axis_name="s",
            num_cores=_NC, num_subcores=_NS),
        scratch_types=[
            pltpu.VMEM((_PP * B,), jnp.int32),
            pltpu.VMEM((2, _CWS), jnp.int32),
            pltpu.VMEM((2, _CWS, 128), jnp.float32),
            pltpu.VMEM((2 * _L,), jnp.int32),
            pltpu.VMEM((2 * _L, 128), jnp.float32),
            pltpu.SemaphoreType.DMA,
            pltpu.SemaphoreType.DMA,
        ],
        compiler_params=pltpu.CompilerParams(needs_layout_passes=False),
    )(_gather_body)


def kernel(x, significance):
    tok = _topk_indices(significance).reshape(B, K)
    tokp = jnp.concatenate(
        [jnp.zeros((B, 1), jnp.int32), tok,
         jnp.zeros((B, _PP - (K + 1)), jnp.int32)], axis=1)
    tok_flat = tokp.T.reshape(_PP * B)
    xa = with_layout_constraint(
        x.transpose(1, 0, 2),
        Layout(major_to_minor=(0, 1, 2), tiling=((4, 128),)))
    xb = with_layout_constraint(
        xa.reshape(N + 1, B, _JT, 128),
        Layout(major_to_minor=(0, 2, 1, 3), tiling=((4, 128),)))
    xc = with_layout_constraint(
        xb.transpose(0, 2, 1, 3),
        Layout(major_to_minor=(0, 1, 2, 3), tiling=((4, 128),)))
    x2 = xc.reshape((N + 1) * _SUB, 128)
    outz = _gather_call()(x2, tok_flat)
    oe = with_layout_constraint(
        outz.reshape(K + 1, _JT, B, 128),
        Layout(major_to_minor=(0, 1, 2, 3), tiling=((4, 128),)))
    of = with_layout_constraint(
        oe.transpose(2, 0, 1, 3),
        Layout(major_to_minor=(1, 2, 0, 3), tiling=((4, 128),)))
    return of.reshape(B, K + 1, D)


# speedup vs baseline: 3.0623x; 1.1010x over previous
import functools

import jax
import jax.numpy as jnp
from jax import lax
from jax.experimental import pallas as pl
from jax.experimental.layout import Layout, with_layout_constraint
from jax.experimental.pallas import tpu as pltpu
import jax.experimental.pallas.tpu_sc as plsc

B = 4
N = 8192
K = 2048
D = 768
ROWS = N // 128
KROWS = K // 128

_NC = 2
_NS = 16
_NW = _NC * _NS
_CHUNK = K // _NW


GROWS = B * ROWS


def _sort_body(sig_ref, out_ref):
    v = sig_ref[...]
    row = lax.broadcasted_iota(jnp.int32, (GROWS, 128), 0)
    col = lax.broadcasted_iota(jnp.int32, (GROWS, 128), 1)
    flat = (row >> 2) * 128 + col
    idx = flat

    k = 2
    while k <= N:
        j = k // 2
        while j >= 1:
            is_lo = (flat & j) == 0
            desc = (flat & k) == 0
            if j >= 128:
                sft = 4 * (j // 128)
                up_v = jnp.roll(v, -sft, axis=0)
                dn_v = jnp.roll(v, sft, axis=0)
                up_i = jnp.roll(idx, -sft, axis=0)
                dn_i = jnp.roll(idx, sft, axis=0)
            else:
                up_v = jnp.roll(v, -j, axis=1)
                dn_v = jnp.roll(v, j, axis=1)
                up_i = jnp.roll(idx, -j, axis=1)
                dn_i = jnp.roll(idx, j, axis=1)
            pv = jnp.where(is_lo, up_v, dn_v)
            pi = jnp.where(is_lo, up_i, dn_i)
            sg = (v > pv) | ((v == pv) & (idx < pi))
            ts = sg ^ is_lo ^ desc
            v = jnp.where(ts, v, pv)
            idx = jnp.where(ts, idx, pi)
            j //= 2
        k *= 2

    out_ref[...] = idx[0:B * KROWS, :] + 1


def _topk_table(significance):
    siga = with_layout_constraint(
        significance.reshape(B, ROWS, 128),
        Layout(major_to_minor=(1, 0, 2), tiling=((4, 128),)))
    sigb = with_layout_constraint(
        siga.transpose(1, 0, 2),
        Layout(major_to_minor=(0, 1, 2), tiling=((4, 128),)))
    sig2 = sigb.reshape(GROWS, 128)
    out = pl.pallas_call(
        _sort_body,
        out_shape=jax.ShapeDtypeStruct((B * KROWS, 128), jnp.int32),
    )(sig2)
    return out.reshape(B * K)


_JT = D // 128
_SUB = _JT * B
_TOT = (K + 1) * _SUB
_CWS = 128
_FULL = (_TOT // _CWS) * _CWS
_PER_W = (_FULL // _CWS) // _NW
_TAIL = _TOT - _FULL
_PP = 2056
_L = 16


def _div24(svec):
    return lax.shift_right_logical(svec * 43691, 20)


def _expand_idx(tok_v, idx_v, buf, off):
    for g in range(_CWS // _L):
        svec = lax.iota(jnp.int32, _L) + (off + g * _L)
        pvec = _div24(svec)
        rvec = svec - pvec * _SUB
        pm = jnp.clip(pvec - 1, 0, K - 1)
        f = ((pm >> 7) << 9) + ((rvec & (B - 1)) << 7) + (pm & 127)
        t = plsc.load_gather(tok_v, [f])
        t = jnp.where(pvec == 0, 0, t)
        idx_v[buf, pl.ds(g * _L, _L)] = t * _SUB + rvec


_NBUF = 3


def _gather_body(x_hbm, tok_hbm, out_hbm, tok_v, idx_v, rows_v,
                 idxt_v, rowst_v, *sems):
    gsem = sems[:_NBUF]
    wsem = sems[_NBUF:]
    wid = lax.axis_index("s") * _NC + lax.axis_index("c")
    pltpu.sync_copy(tok_hbm, tok_v)

    def chunk_off(u):
        return (wid * _PER_W + u) * _CWS

    def start_gather(u):
        bu = u % _NBUF
        _expand_idx(tok_v, idx_v, bu, chunk_off(u))
        return pltpu.async_copy(x_hbm.at[idx_v.at[bu]], rows_v.at[bu],
                                gsem[bu])

    gd = {u: start_gather(u) for u in range(2)}
    wd = {}
    for u in range(_PER_W):
        bu = u % _NBUF
        gd[u].wait()
        wd[u] = pltpu.async_copy(rows_v.at[bu],
                                 out_hbm.at[pl.ds(chunk_off(u), _CWS)],
                                 wsem[bu])
        if u + 2 < _PER_W:
            if u - 1 >= 0:
                wd[u - 1].wait()
            gd[u + 2] = start_gather(u + 2)
    wd[_PER_W - 3].wait()
    wd[_PER_W - 2].wait()
    wd[_PER_W - 1].wait()

    @pl.when(wid == _NW - 1)
    def _tail():
        for g in range(2):
            svec = lax.iota(jnp.int32, _L) + (_FULL + g * _L)
            pvec = _div24(svec)
            rvec = svec - pvec * _SUB
            pm = jnp.clip(pvec - 1, 0, K - 1)
            f = ((pm >> 7) << 9) + ((rvec & (B - 1)) << 7) + (pm & 127)
            t = plsc.load_gather(tok_v, [f])
            t = jnp.where(pvec == 0, 0, t)
            idxt_v[pl.ds(g * _L, _L)] = t * _SUB + rvec
        pltpu.async_copy(x_hbm.at[idxt_v], rowst_v, gsem[0]).wait()
        pltpu.sync_copy(rowst_v.at[pl.ds(0, _TAIL)],
                        out_hbm.at[pl.ds(_FULL, _TAIL)])


@functools.cache
def _gather_call():
    return functools.partial(
        pl.kernel,
        out_type=jax.ShapeDtypeStruct((_TOT, 128), jnp.float32),
        mesh=plsc.VectorSubcoreMesh(
            core_axis_name="c", subcore_axis_name="s",
            num_cores=_NC, num_subcores=_NS),
        scratch_types=[
            pltpu.VMEM((B * K,), jnp.int32),
            pltpu.VMEM((_NBUF, _CWS), jnp.int32),
            pltpu.VMEM((_NBUF, _CWS, 128), jnp.float32),
            pltpu.VMEM((2 * _L,), jnp.int32),
            pltpu.VMEM((2 * _L, 128), jnp.float32),
        ] + [pltpu.SemaphoreType.DMA] * (2 * _NBUF),
        compiler_params=pltpu.CompilerParams(needs_layout_passes=False),
    )(_gather_body)


def kernel(x, significance):
    tok_flat = _topk_table(significance)
    xa = with_layout_constraint(
        x.transpose(1, 0, 2),
        Layout(major_to_minor=(0, 1, 2), tiling=((4, 128),)))
    xb = with_layout_constraint(
        xa.reshape(N + 1, B, _JT, 128),
        Layout(major_to_minor=(0, 2, 1, 3), tiling=((4, 128),)))
    xc = with_layout_constraint(
        xb.transpose(0, 2, 1, 3),
        Layout(major_to_minor=(0, 1, 2, 3), tiling=((4, 128),)))
    x2 = xc.reshape((N + 1) * _SUB, 128)
    outz = _gather_call()(x2, tok_flat)
    oe = with_layout_constraint(
        outz.reshape(K + 1, _JT, B, 128),
        Layout(major_to_minor=(0, 1, 2, 3), tiling=((4, 128),)))
    of = with_layout_constraint(
        oe.transpose(2, 0, 1, 3),
        Layout(major_to_minor=(1, 2, 0, 3), tiling=((4, 128),)))
    return of.reshape(B, K + 1, D)


# speedup vs baseline: 3.0792x; 1.0055x over previous
import functools

import jax
import jax.numpy as jnp
from jax import lax
from jax.experimental import pallas as pl
from jax.experimental.layout import Layout, with_layout_constraint
from jax.experimental.pallas import tpu as pltpu
import jax.experimental.pallas.tpu_sc as plsc

B = 4
N = 8192
K = 2048
D = 768
ROWS = N // 128
KROWS = K // 128

_NC = 2
_NS = 16
_NW = _NC * _NS
_CHUNK = K // _NW


GROWS = B * ROWS


def _sort_body(sig_ref, out_ref):
    v = sig_ref[...]
    row = lax.broadcasted_iota(jnp.int32, (GROWS, 128), 0)
    col = lax.broadcasted_iota(jnp.int32, (GROWS, 128), 1)
    flat = (row >> 2) * 128 + col
    idx = flat

    k = 2
    while k <= N:
        j = k // 2
        while j >= 1:
            is_lo = (flat & j) == 0
            desc = (flat & k) == 0
            if j >= 128:
                sft = 4 * (j // 128)
                up_v = jnp.roll(v, -sft, axis=0)
                dn_v = jnp.roll(v, sft, axis=0)
                up_i = jnp.roll(idx, -sft, axis=0)
                dn_i = jnp.roll(idx, sft, axis=0)
            else:
                up_v = jnp.roll(v, -j, axis=1)
                dn_v = jnp.roll(v, j, axis=1)
                up_i = jnp.roll(idx, -j, axis=1)
                dn_i = jnp.roll(idx, j, axis=1)
            pv = jnp.where(is_lo, up_v, dn_v)
            pi = jnp.where(is_lo, up_i, dn_i)
            sg = (v > pv) | ((v == pv) & (idx < pi))
            ts = sg ^ is_lo ^ desc
            v = jnp.where(ts, v, pv)
            idx = jnp.where(ts, idx, pi)
            j //= 2
        k *= 2

    out_ref[...] = idx[0:B * KROWS, :] + 1


def _topk_table(significance):
    siga = with_layout_constraint(
        significance.reshape(B, ROWS, 128),
        Layout(major_to_minor=(1, 0, 2), tiling=((4, 128),)))
    sigb = with_layout_constraint(
        siga.transpose(1, 0, 2),
        Layout(major_to_minor=(0, 1, 2), tiling=((4, 128),)))
    sig2 = sigb.reshape(GROWS, 128)
    out = pl.pallas_call(
        _sort_body,
        out_shape=jax.ShapeDtypeStruct((B * KROWS, 128), jnp.int32),
    )(sig2)
    return out.reshape(B * K)


_JT = D // 128
_SUB = _JT * B
_TOT = (K + 1) * _SUB
_CWS = 128
_FULL = (_TOT // _CWS) * _CWS
_PER_W = (_FULL // _CWS) // _NW
_TAIL = _TOT - _FULL
_PP = 2056
_L = 16


def _div24(svec):
    return lax.shift_right_logical(svec * 43691, 20)


def _expand_idx(tok_v, idx_v, buf, off):
    for g in range(_CWS // _L):
        svec = lax.iota(jnp.int32, _L) + (off + g * _L)
        pvec = _div24(svec)
        rvec = svec - pvec * _SUB
        pm = jnp.clip(pvec - 1, 0, K - 1)
        f = ((pm >> 7) << 9) + ((rvec & (B - 1)) << 7) + (pm & 127)
        t = plsc.load_gather(tok_v, [f])
        t = jnp.where(pvec == 0, 0, t)
        idx_v[buf, pl.ds(g * _L, _L)] = t * _SUB + rvec


_NBUF = 4


def _gather_body(x_hbm, tok_hbm, out_hbm, tok_v, idx_v, rows_v,
                 idxt_v, rowst_v, *sems):
    gsem = sems[:_NBUF]
    wsem = sems[_NBUF:]
    wid = lax.axis_index("s") * _NC + lax.axis_index("c")
    pltpu.sync_copy(tok_hbm, tok_v)

    def chunk_off(u):
        return (wid * _PER_W + u) * _CWS

    def start_gather(u):
        bu = u % _NBUF
        _expand_idx(tok_v, idx_v, bu, chunk_off(u))
        return pltpu.async_copy(x_hbm.at[idx_v.at[bu]], rows_v.at[bu],
                                gsem[bu])

    gd = {u: start_gather(u) for u in range(_NBUF - 1)}
    wd = {}
    for u in range(_PER_W):
        bu = u % _NBUF
        gd[u].wait()
        wd[u] = pltpu.async_copy(rows_v.at[bu],
                                 out_hbm.at[pl.ds(chunk_off(u), _CWS)],
                                 wsem[bu])
        nxt = u + _NBUF - 1
        if nxt < _PER_W:
            if u - 1 >= 0:
                wd[u - 1].wait()
            gd[nxt] = start_gather(nxt)
    for t in range(max(0, _PER_W - _NBUF), _PER_W):
        wd[t].wait()

    @pl.when(wid == _NW - 1)
    def _tail():
        for g in range(2):
            svec = lax.iota(jnp.int32, _L) + (_FULL + g * _L)
            pvec = _div24(svec)
            rvec = svec - pvec * _SUB
            pm = jnp.clip(pvec - 1, 0, K - 1)
            f = ((pm >> 7) << 9) + ((rvec & (B - 1)) << 7) + (pm & 127)
            t = plsc.load_gather(tok_v, [f])
            t = jnp.where(pvec == 0, 0, t)
            idxt_v[pl.ds(g * _L, _L)] = t * _SUB + rvec
        pltpu.async_copy(x_hbm.at[idxt_v], rowst_v, gsem[0]).wait()
        pltpu.sync_copy(rowst_v.at[pl.ds(0, _TAIL)],
                        out_hbm.at[pl.ds(_FULL, _TAIL)])


@functools.cache
def _gather_call():
    return functools.partial(
        pl.kernel,
        out_type=jax.ShapeDtypeStruct((_TOT, 128), jnp.float32),
        mesh=plsc.VectorSubcoreMesh(
            core_axis_name="c", subcore_axis_name="s",
            num_cores=_NC, num_subcores=_NS),
        scratch_types=[
            pltpu.VMEM((B * K,), jnp.int32),
            pltpu.VMEM((_NBUF, _CWS), jnp.int32),
            pltpu.VMEM((_NBUF, _CWS, 128), jnp.float32),
            pltpu.VMEM((2 * _L,), jnp.int32),
            pltpu.VMEM((2 * _L, 128), jnp.float32),
        ] + [pltpu.SemaphoreType.DMA] * (2 * _NBUF),
        compiler_params=pltpu.CompilerParams(needs_layout_passes=False),
    )(_gather_body)


def kernel(x, significance):
    tok_flat = _topk_table(significance)
    xa = with_layout_constraint(
        x.transpose(1, 0, 2),
        Layout(major_to_minor=(0, 1, 2), tiling=((4, 128),)))
    xb = with_layout_constraint(
        xa.reshape(N + 1, B, _JT, 128),
        Layout(major_to_minor=(0, 2, 1, 3), tiling=((4, 128),)))
    xc = with_layout_constraint(
        xb.transpose(0, 2, 1, 3),
        Layout(major_to_minor=(0, 1, 2, 3), tiling=((4, 128),)))
    x2 = xc.reshape((N + 1) * _SUB, 128)
    outz = _gather_call()(x2, tok_flat)
    oe = with_layout_constraint(
        outz.reshape(K + 1, _JT, B, 128),
        Layout(major_to_minor=(0, 1, 2, 3), tiling=((4, 128),)))
    of = with_layout_constraint(
        oe.transpose(2, 0, 1, 3),
        Layout(major_to_minor=(1, 2, 0, 3), tiling=((4, 128),)))
    return of.reshape(B, K + 1, D)


# speedup vs baseline: 3.0889x; 1.0032x over previous
import functools

import jax
import jax.numpy as jnp
from jax import lax
from jax.experimental import pallas as pl
from jax.experimental.layout import Layout, with_layout_constraint
from jax.experimental.pallas import tpu as pltpu
import jax.experimental.pallas.tpu_sc as plsc

B = 4
N = 8192
K = 2048
D = 768
ROWS = N // 128
KROWS = K // 128

_NC = 2
_NS = 16
_NW = _NC * _NS


GROWS = B * ROWS


def _sort_body(sig_ref, out_ref):
    v = sig_ref[...]
    row = lax.broadcasted_iota(jnp.int32, (GROWS, 128), 0)
    col = lax.broadcasted_iota(jnp.int32, (GROWS, 128), 1)
    flat = (row >> 2) * 128 + col
    idx = flat

    k = 2
    while k <= N:
        j = k // 2
        while j >= 1:
            is_lo = (flat & j) == 0
            desc = (flat & k) == 0
            if j >= 128:
                sft = 4 * (j // 128)
                up_v = jnp.roll(v, -sft, axis=0)
                dn_v = jnp.roll(v, sft, axis=0)
                up_i = jnp.roll(idx, -sft, axis=0)
                dn_i = jnp.roll(idx, sft, axis=0)
            else:
                up_v = jnp.roll(v, -j, axis=1)
                dn_v = jnp.roll(v, j, axis=1)
                up_i = jnp.roll(idx, -j, axis=1)
                dn_i = jnp.roll(idx, j, axis=1)
            pv = jnp.where(is_lo, up_v, dn_v)
            pi = jnp.where(is_lo, up_i, dn_i)
            sg = (v > pv) | ((v == pv) & (idx < pi))
            ts = sg ^ is_lo ^ desc
            v = jnp.where(ts, v, pv)
            idx = jnp.where(ts, idx, pi)
            j //= 2
        k *= 2

    out_ref[...] = idx[0:B * KROWS, :] + 1


def _topk_table(significance):
    siga = with_layout_constraint(
        significance.reshape(B, ROWS, 128),
        Layout(major_to_minor=(1, 0, 2), tiling=((4, 128),)))
    sigb = with_layout_constraint(
        siga.transpose(1, 0, 2),
        Layout(major_to_minor=(0, 1, 2), tiling=((4, 128),)))
    sig2 = sigb.reshape(GROWS, 128)
    out = pl.pallas_call(
        _sort_body,
        out_shape=jax.ShapeDtypeStruct((B * KROWS, 128), jnp.int32),
    )(sig2)
    return out.reshape(B * K)


_JT = D // 128
_SUB = _JT * B
_TOT = (K + 1) * _SUB
_CWS = 128
_FULL = (_TOT // _CWS) * _CWS
_PER_W = (_FULL // _CWS) // _NW
_TAIL = _TOT - _FULL
_L = 16


def _div24(svec):
    return lax.shift_right_logical(svec * 43691, 20)


def _expand_idx(tok_v, idx_v, buf, off):
    for g in range(_CWS // _L):
        svec = lax.iota(jnp.int32, _L) + (off + g * _L)
        pvec = _div24(svec)
        rvec = svec - pvec * _SUB
        pm = jnp.clip(pvec - 1, 0, K - 1)
        f = ((pm >> 7) << 9) + ((rvec & (B - 1)) << 7) + (pm & 127)
        t = plsc.load_gather(tok_v, [f])
        t = jnp.where(pvec == 0, 0, t)
        idx_v[buf, pl.ds(g * _L, _L)] = t * _SUB + rvec


_NBUF = 4


def _gather_body(x_hbm, tok_hbm, out_hbm, tok_v, idx_v, rows_v,
                 idxt_v, rowst_v, *sems):
    gsem = sems[:_NBUF]
    wsem = sems[_NBUF:]
    wid = lax.axis_index("s") * _NC + lax.axis_index("c")
    pltpu.sync_copy(tok_hbm, tok_v)

    def chunk_off(u):
        return (wid * _PER_W + u) * _CWS

    def start_gather(u):
        bu = u % _NBUF
        _expand_idx(tok_v, idx_v, bu, chunk_off(u))
        return pltpu.async_copy(x_hbm.at[idx_v.at[bu]], rows_v.at[bu],
                                gsem[bu])

    gd = {u: start_gather(u) for u in range(_NBUF - 1)}
    wd = {}
    for u in range(_PER_W):
        bu = u % _NBUF
        gd[u].wait()
        wd[u] = pltpu.async_copy(rows_v.at[bu],
                                 out_hbm.at[pl.ds(chunk_off(u), _CWS)],
                                 wsem[bu])
        nxt = u + _NBUF - 1
        if nxt < _PER_W:
            if u - 1 >= 0:
                wd[u - 1].wait()
            gd[nxt] = start_gather(nxt)
    for t in range(max(0, _PER_W - _NBUF), _PER_W):
        wd[t].wait()

    @pl.when(wid == _NW - 1)
    def _tail():
        for g in range(2):
            svec = lax.iota(jnp.int32, _L) + (_FULL + g * _L)
            pvec = _div24(svec)
            rvec = svec - pvec * _SUB
            pm = jnp.clip(pvec - 1, 0, K - 1)
            f = ((pm >> 7) << 9) + ((rvec & (B - 1)) << 7) + (pm & 127)
            t = plsc.load_gather(tok_v, [f])
            t = jnp.where(pvec == 0, 0, t)
            idxt_v[pl.ds(g * _L, _L)] = t * _SUB + rvec
        pltpu.async_copy(x_hbm.at[idxt_v], rowst_v, gsem[0]).wait()
        pltpu.sync_copy(rowst_v.at[pl.ds(0, _TAIL)],
                        out_hbm.at[pl.ds(_FULL, _TAIL)])


@functools.cache
def _gather_call():
    return functools.partial(
        pl.kernel,
        out_type=jax.ShapeDtypeStruct((_TOT, 128), jnp.float32),
        mesh=plsc.VectorSubcoreMesh(
            core_axis_name="c", subcore_axis_name="s",
            num_cores=_NC, num_subcores=_NS),
        scratch_types=[
            pltpu.VMEM((B * K,), jnp.int32),
            pltpu.VMEM((_NBUF, _CWS), jnp.int32),
            pltpu.VMEM((_NBUF, _CWS, 128), jnp.float32),
            pltpu.VMEM((2 * _L,), jnp.int32),
            pltpu.VMEM((2 * _L, 128), jnp.float32),
        ] + [pltpu.SemaphoreType.DMA] * (2 * _NBUF),
        compiler_params=pltpu.CompilerParams(needs_layout_passes=False),
    )(_gather_body)


def kernel(x, significance):
    tok_flat = _topk_table(significance)
    xa = with_layout_constraint(
        x.transpose(1, 0, 2),
        Layout(major_to_minor=(0, 1, 2), tiling=((4, 128),)))
    xb = with_layout_constraint(
        xa.reshape(N + 1, B, _JT, 128),
        Layout(major_to_minor=(0, 2, 1, 3), tiling=((4, 128),)))
    xc = with_layout_constraint(
        xb.transpose(0, 2, 1, 3),
        Layout(major_to_minor=(0, 1, 2, 3), tiling=((4, 128),)))
    x2 = xc.reshape((N + 1) * _SUB, 128)
    outz = _gather_call()(x2, tok_flat)
    oe = with_layout_constraint(
        outz.reshape(K + 1, _JT, B, 128),
        Layout(major_to_minor=(0, 1, 2, 3), tiling=((4, 128),)))
    of = with_layout_constraint(
        oe.transpose(2, 0, 1, 3),
        Layout(major_to_minor=(1, 2, 0, 3), tiling=((4, 128),)))
    return of.reshape(B, K + 1, D)
